# Initial kernel scaffold; baseline (speedup 1.0000x reference)
import jax
import jax.numpy as jnp
from jax.experimental import pallas as pl


def kernel(features, pos, pos_center, W1, b1, W2, b2, Wg1, bg1, Wg2, bg2, Wd1, bd1, Wd2, bd2, Wbp, bbp, Wq, Wk, Wv):
    raise NotImplementedError("write your pallas kernel here")


# speedup vs baseline: 7.7748x; 7.7748x over previous
import functools

import jax
import jax.numpy as jnp
import numpy as np
from jax import lax
from jax.experimental import pallas as pl
from jax.experimental.pallas import tpu as pltpu
from jax.experimental.pallas import tpu_sc as plsc

B, N, D, K = 4, 1000, 256, 16
NP = 1024
RB = 256
Q = 128

SC_NC, SC_NS = 2, 16
SC_NW = SC_NC * SC_NS
GROWS = B * NP * K
ROWS_PER_W = GROWS // SC_NW
CHUNK = 128


def _prep_body(wk, wg1, wq, wd2, bg1, bd2, wkg, wqg, m, cb):
    g1 = wg1[...]
    wkg[...] = jnp.dot(wk[...], g1, preferred_element_type=jnp.float32)
    wqg[...] = jnp.dot(wq[...], g1, preferred_element_type=jnp.float32)
    m[...] = jnp.dot(wd2[...], g1, preferred_element_type=jnp.float32)
    cb[...] = bg1[...] + jnp.dot(bd2[...], g1,
                                 preferred_element_type=jnp.float32)


def _prep(Wk, Wg1, Wq, Wd2, bg1, bd2):
    f = jax.ShapeDtypeStruct((D, D), jnp.float32)
    r = jax.ShapeDtypeStruct((1, D), jnp.float32)
    return pl.pallas_call(
        _prep_body,
        out_shape=(f, f, f, r),
    )(Wk, Wg1, Wq, Wd2, bg1.reshape(1, D), bd2.reshape(1, D))


def _topk_body(pc, pct, wbp, bbp, knn_out, gidx_out):
    b = pl.program_id(0)
    pcb = pc[0]
    pctb = pct[0]
    cxr = pcb[:, 0:1]
    cyr = pcb[:, 1:2]
    hwr = 0.5 * pcb[:, 2:3]
    hhr = 0.5 * pcb[:, 3:4]
    x1r, y1r = cxr - hwr, cyr - hhr
    x2r, y2r = cxr + hwr, cyr + hhr
    cxc = pctb[0:1, :]
    cyc = pctb[1:2, :]
    hwc = 0.5 * pctb[2:3, :]
    hhc = 0.5 * pctb[3:4, :]
    x1c, y1c = cxc - hwc, cyc - hhc
    x2c, y2c = cxc + hwc, cyc + hhc

    dx = cxr - cxc
    dy = cyr - cyc
    dis = jnp.sqrt(dx * dx + dy * dy)
    ow = jnp.clip(jnp.minimum(x2r, x2c) - jnp.maximum(x1r, x1c), 0.0, None)
    oh = jnp.clip(jnp.minimum(y2r, y2c) - jnp.maximum(y1r, y1c), 0.0, None)
    uw = jnp.clip(jnp.maximum(x2r, x2c) - jnp.minimum(x1r, x1c), 0.0, None)
    uh = jnp.clip(jnp.maximum(y2r, y2c) - jnp.minimum(y1r, y1c), 0.0, None)
    iou = (ow * oh) / (uw * uh + 1e-06)
    s = jnp.concatenate([dis, iou], axis=0)
    ri = lax.broadcasted_iota(jnp.int32, (RB, 2 * RB), 0)
    ci = lax.broadcasted_iota(jnp.int32, (RB, 2 * RB), 1)
    wc = jnp.where(ci == ri, wbp[0], 0.0) + jnp.where(ci == ri + RB, wbp[1], 0.0)
    vals = jnp.dot(wc, s, preferred_element_type=jnp.float32) + bbp[0]

    lane = lax.broadcasted_iota(jnp.int32, (RB, NP), 1)
    inf = jnp.float32(np.inf)
    vals = jnp.where(lane >= N, inf, vals)
    cols = []
    for _ in range(K):
        m = jnp.min(vals, axis=1, keepdims=True)
        cand = jnp.where(vals == m, lane, jnp.int32(1 << 30))
        j = jnp.min(cand, axis=1, keepdims=True)
        vals = jnp.where(lane == j, inf, vals)
        cols.append(j)
    knn = jnp.concatenate(cols, axis=1)
    knn_out[0] = knn
    gidx_out[0] = knn + b * NP


def _topk(pc_pad, pct_pad, wbp, bbp):
    grid = (B, NP // RB)
    return pl.pallas_call(
        _topk_body,
        grid=grid,
        in_specs=[
            pl.BlockSpec((1, RB, 4), lambda b, r: (b, r, 0)),
            pl.BlockSpec((1, 8, NP), lambda b, r: (b, 0, 0)),
            pl.BlockSpec(memory_space=pltpu.SMEM),
            pl.BlockSpec(memory_space=pltpu.SMEM),
        ],
        out_specs=(
            pl.BlockSpec((1, RB, K), lambda b, r: (b, r, 0)),
            pl.BlockSpec((1, RB, K), lambda b, r: (b, r, 0)),
        ),
        out_shape=(
            jax.ShapeDtypeStruct((B, N, K), jnp.int32),
            jax.ShapeDtypeStruct((B, NP, K), jnp.int32),
        ),
    )(pc_pad, pct_pad, wbp, bbp)


def _pernode_body(feat, pos, w1, b1, wd1, wv, wkg, wqg, cb, t_out, p_out):
    x = jnp.dot(feat[0], w1[...], preferred_element_type=jnp.float32) + b1[...]
    pd = jnp.dot(pos[0], wd1[...], preferred_element_type=jnp.float32)
    d = jnp.dot(x, wkg[...], preferred_element_type=jnp.float32)
    v = jnp.dot(x, wv[...], preferred_element_type=jnp.float32)
    c = jnp.dot(x, wqg[...], preferred_element_type=jnp.float32) + cb[...]
    t_out[0] = jnp.concatenate([pd, d, v], axis=1)
    p_out[0] = jnp.concatenate([pd, c], axis=1)


def _pernode(feat_pad, pos_pad, W1, b1, Wd1, Wv, Wkg, Wqg, cb):
    grid = (B, NP // RB)
    full = lambda b, r: (0, 0)
    return pl.pallas_call(
        _pernode_body,
        grid=grid,
        in_specs=[
            pl.BlockSpec((1, RB, D), lambda b, r: (b, r, 0)),
            pl.BlockSpec((1, RB, D), lambda b, r: (b, r, 0)),
            pl.BlockSpec((D, D), full),
            pl.BlockSpec((1, D), full),
            pl.BlockSpec((D, D), full),
            pl.BlockSpec((D, D), full),
            pl.BlockSpec((D, D), full),
            pl.BlockSpec((D, D), full),
            pl.BlockSpec((1, D), full),
        ],
        out_specs=(
            pl.BlockSpec((1, RB, 3 * D), lambda b, r: (b, r, 0)),
            pl.BlockSpec((1, RB, 2 * D), lambda b, r: (b, r, 0)),
        ),
        out_shape=(
            jax.ShapeDtypeStruct((B, NP, 3 * D), jnp.float32),
            jax.ShapeDtypeStruct((B, NP, 2 * D), jnp.float32),
        ),
    )(feat_pad, pos_pad, W1, b1.reshape(1, D), Wd1, Wv, Wkg, Wqg, cb)


def _sc_gather_body(table_hbm, idx_hbm, out_hbm, idx_v, rows_v, sem):
    wid = lax.axis_index("s") * SC_NC + lax.axis_index("c")
    base = wid * ROWS_PER_W

    def body(j, carry):
        off = base + j * CHUNK
        pltpu.sync_copy(idx_hbm.at[pl.ds(off, CHUNK)], idx_v)
        pltpu.async_copy(table_hbm.at[idx_v], rows_v, sem).wait()
        pltpu.sync_copy(rows_v, out_hbm.at[pl.ds(off, CHUNK)])
        return carry

    lax.fori_loop(0, ROWS_PER_W // CHUNK, body, 0)


def _sc_gather(table_flat, gidx_flat):
    mesh = plsc.VectorSubcoreMesh(core_axis_name="c", subcore_axis_name="s")
    kfn = functools.partial(
        pl.kernel,
        mesh=mesh,
        out_type=jax.ShapeDtypeStruct((GROWS, 3 * D), jnp.float32),
        scratch_types=[
            pltpu.VMEM((CHUNK,), jnp.int32),
            pltpu.VMEM((CHUNK, 3 * D), jnp.float32),
            pltpu.SemaphoreType.DMA,
        ],
    )(_sc_gather_body)
    return kfn(table_flat, gidx_flat)


def _main_body(g, p, feat, wcat, wg2, w2, bd1, bd2, bg2, b2, out):
    gb = g[0]
    pdg = gb[:, 0:D].reshape(Q, K, D)
    dg = gb[:, D:2 * D].reshape(Q, K, D)
    vg = gb[:, 2 * D:3 * D].reshape(Q, K, D)
    pb = p[0]
    pdq = pb[:, 0:D]
    cq = pb[:, D:2 * D]

    h = pdq[:, None, :] - pdg + bd1[...]
    r = jnp.maximum(h, 0.0).reshape(Q * K, D)
    pa = jnp.dot(r, wcat[...], preferred_element_type=jnp.float32)
    pe = pa[:, 0:D].reshape(Q, K, D) + bd2[...]
    a1 = pa[:, D:2 * D].reshape(Q, K, D) + cq[:, None, :] - dg
    a2 = jnp.dot(jnp.maximum(a1, 0.0).reshape(Q * K, D), wg2[...],
                 preferred_element_type=jnp.float32)
    a2 = a2.reshape(Q, K, D) + bg2[...]

    z = a2 * jnp.float32(1.0 / 16.0)
    zmax = jnp.max(z, axis=1, keepdims=True)
    e = jnp.exp(z - zmax)
    attn = e / jnp.sum(e, axis=1, keepdims=True)
    o = jnp.sum(attn * (vg + pe), axis=1)
    res = jnp.dot(o, w2[...], preferred_element_type=jnp.float32)
    out[0] = res + b2[...] + feat[0]


def _main(g_rows, p_arr, feat_pad, Wcat, Wg2, W2, bd1, bd2, bg2, b2):
    grid = (B, NP // Q)
    full = lambda b, q: (0, 0)
    g4 = g_rows.reshape(B, NP * K, 3 * D)
    return pl.pallas_call(
        _main_body,
        grid=grid,
        in_specs=[
            pl.BlockSpec((1, Q * K, 3 * D), lambda b, q: (b, q, 0)),
            pl.BlockSpec((1, Q, 2 * D), lambda b, q: (b, q, 0)),
            pl.BlockSpec((1, Q, D), lambda b, q: (b, q, 0)),
            pl.BlockSpec((D, 2 * D), full),
            pl.BlockSpec((D, D), full),
            pl.BlockSpec((D, D), full),
            pl.BlockSpec((1, D), full),
            pl.BlockSpec((1, D), full),
            pl.BlockSpec((1, D), full),
            pl.BlockSpec((1, D), full),
        ],
        out_specs=pl.BlockSpec((1, Q, D), lambda b, q: (b, q, 0)),
        out_shape=jax.ShapeDtypeStruct((B, NP, D), jnp.float32),
    )(g4, p_arr, feat_pad, Wcat, Wg2, W2,
      bd1.reshape(1, D), bd2.reshape(1, D), bg2.reshape(1, D),
      b2.reshape(1, D))


def kernel(features, pos, pos_center, W1, b1, W2, b2, Wg1, bg1, Wg2, bg2,
           Wd1, bd1, Wd2, bd2, Wbp, bbp, Wq, Wk, Wv):
    pad_n = [(0, 0), (0, NP - N), (0, 0)]
    feat_pad = jnp.pad(features, pad_n)
    pos_pad = jnp.pad(pos, pad_n)
    pc_pad = jnp.pad(pos_center, pad_n)
    pct_pad = jnp.pad(pos_center.transpose(0, 2, 1), [(0, 0), (0, 4), (0, NP - N)])

    Wkg, Wqg, M, cb = _prep(Wk, Wg1, Wq, Wd2, bg1, bd2)
    knn_idx, gidx = _topk(pc_pad, pct_pad, Wbp.reshape(2), bbp)
    T, P = _pernode(feat_pad, pos_pad, W1, b1, Wd1, Wv, Wkg, Wqg, cb)

    g_rows = _sc_gather(T.reshape(B * NP, 3 * D), gidx.reshape(GROWS))
    Wcat = jnp.concatenate([Wd2, M], axis=1)
    res = _main(g_rows, P, feat_pad, Wcat, Wg2, W2, bd1, bd2, bg2, b2)
    return res[:, :N, :], knn_idx


# speedup vs baseline: 9.5975x; 1.2344x over previous
import functools

import jax
import jax.numpy as jnp
import numpy as np
from jax import lax
from jax.experimental import pallas as pl
from jax.experimental.pallas import tpu as pltpu
from jax.experimental.pallas import tpu_sc as plsc

B, N, D, K = 4, 1000, 256, 16
NP = 1024
RB = 256
Q = 128

SC_NC, SC_NS = 2, 16
SC_NW = SC_NC * SC_NS
GROWS = B * NP * K
HW = 3 * D // 2
ROWS_PER_W = GROWS // SC_NW
CHUNK = 128


def _prep_body(wk, wg1, wq, wd2, bg1, bd2, wkg, wqg, m, cb):
    g1 = wg1[...]
    wkg[...] = jnp.dot(wk[...], g1, preferred_element_type=jnp.float32)
    wqg[...] = jnp.dot(wq[...], g1, preferred_element_type=jnp.float32)
    m[...] = jnp.dot(wd2[...], g1, preferred_element_type=jnp.float32)
    cb[...] = bg1[...] + jnp.dot(bd2[...], g1,
                                 preferred_element_type=jnp.float32)


def _prep(Wk, Wg1, Wq, Wd2, bg1, bd2):
    f = jax.ShapeDtypeStruct((D, D), jnp.float32)
    r = jax.ShapeDtypeStruct((1, D), jnp.float32)
    return pl.pallas_call(
        _prep_body,
        out_shape=(f, f, f, r),
    )(Wk, Wg1, Wq, Wd2, bg1.reshape(1, D), bd2.reshape(1, D))


def _topk_body(pc, pct, wbp, bbp, knn_out, gidx_out):
    b = pl.program_id(0)
    pcb = pc[0]
    pctb = pct[0]
    cxr = pcb[:, 0:1]
    cyr = pcb[:, 1:2]
    hwr = 0.5 * pcb[:, 2:3]
    hhr = 0.5 * pcb[:, 3:4]
    x1r, y1r = cxr - hwr, cyr - hhr
    x2r, y2r = cxr + hwr, cyr + hhr
    cxc = pctb[0:1, :]
    cyc = pctb[1:2, :]
    hwc = 0.5 * pctb[2:3, :]
    hhc = 0.5 * pctb[3:4, :]
    x1c, y1c = cxc - hwc, cyc - hhc
    x2c, y2c = cxc + hwc, cyc + hhc

    dx = cxr - cxc
    dy = cyr - cyc
    dis = jnp.sqrt(dx * dx + dy * dy)
    ow = jnp.clip(jnp.minimum(x2r, x2c) - jnp.maximum(x1r, x1c), 0.0, None)
    oh = jnp.clip(jnp.minimum(y2r, y2c) - jnp.maximum(y1r, y1c), 0.0, None)
    uw = jnp.clip(jnp.maximum(x2r, x2c) - jnp.minimum(x1r, x1c), 0.0, None)
    uh = jnp.clip(jnp.maximum(y2r, y2c) - jnp.minimum(y1r, y1c), 0.0, None)
    iou = (ow * oh) / (uw * uh + 1e-06)
    s = jnp.concatenate([dis, iou], axis=0)
    ri = lax.broadcasted_iota(jnp.int32, (RB, 2 * RB), 0)
    ci = lax.broadcasted_iota(jnp.int32, (RB, 2 * RB), 1)
    wc = jnp.where(ci == ri, wbp[0], 0.0) + jnp.where(ci == ri + RB, wbp[1], 0.0)
    vals = jnp.dot(wc, s, preferred_element_type=jnp.float32) + bbp[0]

    lane = lax.broadcasted_iota(jnp.int32, (RB, NP), 1)
    inf = jnp.float32(np.inf)
    vals = jnp.where(lane >= N, inf, vals)
    cols = []
    for _ in range(K):
        m = jnp.min(vals, axis=1, keepdims=True)
        cand = jnp.where(vals == m, lane, jnp.int32(1 << 30))
        j = jnp.min(cand, axis=1, keepdims=True)
        vals = jnp.where(lane == j, inf, vals)
        cols.append(j)
    knn = jnp.concatenate(cols, axis=1)
    knn_out[0] = knn
    gidx_out[0] = knn + b * NP


def _topk(pc_pad, pct_pad, wbp, bbp):
    grid = (B, NP // RB)
    return pl.pallas_call(
        _topk_body,
        grid=grid,
        in_specs=[
            pl.BlockSpec((1, RB, 4), lambda b, r: (b, r, 0)),
            pl.BlockSpec((1, 8, NP), lambda b, r: (b, 0, 0)),
            pl.BlockSpec(memory_space=pltpu.SMEM),
            pl.BlockSpec(memory_space=pltpu.SMEM),
        ],
        out_specs=(
            pl.BlockSpec((1, RB, K), lambda b, r: (b, r, 0)),
            pl.BlockSpec((1, RB, K), lambda b, r: (b, r, 0)),
        ),
        out_shape=(
            jax.ShapeDtypeStruct((B, N, K), jnp.int32),
            jax.ShapeDtypeStruct((B, NP, K), jnp.int32),
        ),
    )(pc_pad, pct_pad, wbp, bbp)


def _pernode_body(feat, pos, w1, b1, wd1, wv, wkg, wqg, cb, t_out, p_out):
    x = jnp.dot(feat[0], w1[...], preferred_element_type=jnp.float32) + b1[...]
    pd = jnp.dot(pos[0], wd1[...], preferred_element_type=jnp.float32)
    d = jnp.dot(x, wkg[...], preferred_element_type=jnp.float32)
    v = jnp.dot(x, wv[...], preferred_element_type=jnp.float32)
    c = jnp.dot(x, wqg[...], preferred_element_type=jnp.float32) + cb[...]
    t = jnp.concatenate([pd, d, v], axis=1)
    tl = t[:, :HW].astype(jnp.bfloat16).astype(jnp.float32)
    th = t[:, HW:].astype(jnp.bfloat16).astype(jnp.float32)
    word = (lax.bitcast_convert_type(th, jnp.uint32)
            | (lax.bitcast_convert_type(tl, jnp.uint32) >> 16))
    t_out[0] = lax.bitcast_convert_type(word, jnp.int32)
    p_out[0] = jnp.concatenate([pd, c], axis=1)


def _pernode(feat_pad, pos_pad, W1, b1, Wd1, Wv, Wkg, Wqg, cb):
    grid = (B, NP // RB)
    full = lambda b, r: (0, 0)
    return pl.pallas_call(
        _pernode_body,
        grid=grid,
        in_specs=[
            pl.BlockSpec((1, RB, D), lambda b, r: (b, r, 0)),
            pl.BlockSpec((1, RB, D), lambda b, r: (b, r, 0)),
            pl.BlockSpec((D, D), full),
            pl.BlockSpec((1, D), full),
            pl.BlockSpec((D, D), full),
            pl.BlockSpec((D, D), full),
            pl.BlockSpec((D, D), full),
            pl.BlockSpec((D, D), full),
            pl.BlockSpec((1, D), full),
        ],
        out_specs=(
            pl.BlockSpec((1, RB, HW), lambda b, r: (b, r, 0)),
            pl.BlockSpec((1, RB, 2 * D), lambda b, r: (b, r, 0)),
        ),
        out_shape=(
            jax.ShapeDtypeStruct((B, NP, HW), jnp.int32),
            jax.ShapeDtypeStruct((B, NP, 2 * D), jnp.float32),
        ),
    )(feat_pad, pos_pad, W1, b1.reshape(1, D), Wd1, Wv, Wkg, Wqg, cb)


def _sc_gather_body(table_hbm, idx_hbm, out_hbm,
                    idx0, idx1, rows0, rows1, si0, si1, sg, sw0, sw1):
    wid = lax.axis_index("s") * SC_NC + lax.axis_index("c")
    base = wid * ROWS_PER_W
    nch = ROWS_PER_W // CHUNK
    idx_v = (idx0, idx1)
    rows_v = (rows0, rows1)
    si = (si0, si1)
    sw = (sw0, sw1)

    pltpu.async_copy(idx_hbm.at[pl.ds(base, CHUNK)], idx_v[0], si[0])
    for j in range(nch):
        p = j % 2
        si_c = pltpu.make_async_copy(
            idx_hbm.at[pl.ds(base + j * CHUNK, CHUNK)], idx_v[p], si[p])
        si_c.wait()
        if j + 1 < nch:
            pltpu.async_copy(idx_hbm.at[pl.ds(base + (j + 1) * CHUNK, CHUNK)],
                             idx_v[(j + 1) % 2], si[(j + 1) % 2])
        if j >= 2:
            pltpu.make_async_copy(
                rows_v[p], out_hbm.at[pl.ds(base + (j - 2) * CHUNK, CHUNK)],
                sw[p]).wait()
        pltpu.async_copy(table_hbm.at[idx_v[p]], rows_v[p], sg).wait()
        pltpu.async_copy(rows_v[p], out_hbm.at[pl.ds(base + j * CHUNK, CHUNK)],
                         sw[p])
    for j in range(max(nch - 2, 0), nch):
        p = j % 2
        pltpu.make_async_copy(
            rows_v[p], out_hbm.at[pl.ds(base + j * CHUNK, CHUNK)], sw[p]).wait()


def _sc_gather(table_flat, gidx_flat):
    mesh = plsc.VectorSubcoreMesh(core_axis_name="c", subcore_axis_name="s")
    kfn = functools.partial(
        pl.kernel,
        mesh=mesh,
        out_type=jax.ShapeDtypeStruct((GROWS, HW), jnp.int32),
        scratch_types=[
            pltpu.VMEM((CHUNK,), jnp.int32),
            pltpu.VMEM((CHUNK,), jnp.int32),
            pltpu.VMEM((CHUNK, HW), jnp.int32),
            pltpu.VMEM((CHUNK, HW), jnp.int32),
            pltpu.SemaphoreType.DMA,
            pltpu.SemaphoreType.DMA,
            pltpu.SemaphoreType.DMA,
            pltpu.SemaphoreType.DMA,
            pltpu.SemaphoreType.DMA,
        ],
    )(_sc_gather_body)
    return kfn(table_flat, gidx_flat)


def _main_body(g, p, feat, wcat, wg2, w2, bd1, bd2, bg2, b2, out):
    gw = lax.bitcast_convert_type(g[0], jnp.uint32)
    lo = lax.bitcast_convert_type(gw << 16, jnp.float32)
    hi = lax.bitcast_convert_type(gw & jnp.uint32(0xFFFF0000), jnp.float32)
    pdg = lo[:, 0:D].reshape(Q, K, D)
    dg = jnp.concatenate([lo[:, D:HW], hi[:, 0:HW - D]], axis=1).reshape(Q, K, D)
    vg = hi[:, HW - D:HW].reshape(Q, K, D)
    pb = p[0]
    pdq = pb[:, 0:D]
    cq = pb[:, D:2 * D]

    h = pdq[:, None, :] - pdg + bd1[...]
    r = jnp.maximum(h, 0.0).reshape(Q * K, D)
    pa = jnp.dot(r, wcat[...], preferred_element_type=jnp.float32)
    pe = pa[:, 0:D].reshape(Q, K, D) + bd2[...]
    a1 = pa[:, D:2 * D].reshape(Q, K, D) + cq[:, None, :] - dg
    a2 = jnp.dot(jnp.maximum(a1, 0.0).reshape(Q * K, D), wg2[...],
                 preferred_element_type=jnp.float32)
    a2 = a2.reshape(Q, K, D) + bg2[...]

    z = a2 * jnp.float32(1.0 / 16.0)
    zmax = jnp.max(z, axis=1, keepdims=True)
    e = jnp.exp(z - zmax)
    attn = e / jnp.sum(e, axis=1, keepdims=True)
    o = jnp.sum(attn * (vg + pe), axis=1)
    res = jnp.dot(o, w2[...], preferred_element_type=jnp.float32)
    out[0] = res + b2[...] + feat[0]


def _main(g_rows, p_arr, feat_pad, Wcat, Wg2, W2, bd1, bd2, bg2, b2):
    grid = (B, NP // Q)
    full = lambda b, q: (0, 0)
    g4 = g_rows.reshape(B, NP * K, HW)
    return pl.pallas_call(
        _main_body,
        grid=grid,
        in_specs=[
            pl.BlockSpec((1, Q * K, HW), lambda b, q: (b, q, 0)),
            pl.BlockSpec((1, Q, 2 * D), lambda b, q: (b, q, 0)),
            pl.BlockSpec((1, Q, D), lambda b, q: (b, q, 0)),
            pl.BlockSpec((D, 2 * D), full),
            pl.BlockSpec((D, D), full),
            pl.BlockSpec((D, D), full),
            pl.BlockSpec((1, D), full),
            pl.BlockSpec((1, D), full),
            pl.BlockSpec((1, D), full),
            pl.BlockSpec((1, D), full),
        ],
        out_specs=pl.BlockSpec((1, Q, D), lambda b, q: (b, q, 0)),
        out_shape=jax.ShapeDtypeStruct((B, NP, D), jnp.float32),
    )(g4, p_arr, feat_pad, Wcat, Wg2, W2,
      bd1.reshape(1, D), bd2.reshape(1, D), bg2.reshape(1, D),
      b2.reshape(1, D))


def kernel(features, pos, pos_center, W1, b1, W2, b2, Wg1, bg1, Wg2, bg2,
           Wd1, bd1, Wd2, bd2, Wbp, bbp, Wq, Wk, Wv):
    pad_n = [(0, 0), (0, NP - N), (0, 0)]
    feat_pad = jnp.pad(features, pad_n)
    pos_pad = jnp.pad(pos, pad_n)
    pc_pad = jnp.pad(pos_center, pad_n)
    pct_pad = jnp.pad(pos_center.transpose(0, 2, 1), [(0, 0), (0, 4), (0, NP - N)])

    Wkg, Wqg, M, cb = _prep(Wk, Wg1, Wq, Wd2, bg1, bd2)
    knn_idx, gidx = _topk(pc_pad, pct_pad, Wbp.reshape(2), bbp)
    T, P = _pernode(feat_pad, pos_pad, W1, b1, Wd1, Wv, Wkg, Wqg, cb)

    g_rows = _sc_gather(T.reshape(B * NP, HW), gidx.reshape(GROWS))
    Wcat = jnp.concatenate([Wd2, M], axis=1)
    res = _main(g_rows, P, feat_pad, Wcat, Wg2, W2, bd1, bd2, bg2, b2)
    return res[:, :N, :], knn_idx


# speedup vs baseline: 9.7442x; 1.0153x over previous
import functools

import jax
import jax.numpy as jnp
import numpy as np
from jax import lax
from jax.experimental import pallas as pl
from jax.experimental.pallas import tpu as pltpu
from jax.experimental.pallas import tpu_sc as plsc

B, N, D, K = 4, 1000, 256, 16
NP = 1024
RB = 256
Q = 128

SC_NC, SC_NS = 2, 16
SC_NW = SC_NC * SC_NS
GROWS = B * NP * K
HW = 3 * D // 2
BROWS = NP * K
ROWS_PER_W = BROWS // SC_NW
CHUNK = 128


def _prep_body(wk, wg1, wq, wd2, bg1, bd2, wkg, wqg, m, cb):
    g1 = wg1[...]
    wkg[...] = jnp.dot(wk[...], g1, preferred_element_type=jnp.float32)
    wqg[...] = jnp.dot(wq[...], g1, preferred_element_type=jnp.float32)
    m[...] = jnp.dot(wd2[...], g1, preferred_element_type=jnp.float32)
    cb[...] = bg1[...] + jnp.dot(bd2[...], g1,
                                 preferred_element_type=jnp.float32)


def _prep(Wk, Wg1, Wq, Wd2, bg1, bd2):
    f = jax.ShapeDtypeStruct((D, D), jnp.float32)
    r = jax.ShapeDtypeStruct((1, D), jnp.float32)
    return pl.pallas_call(
        _prep_body,
        out_shape=(f, f, f, r),
    )(Wk, Wg1, Wq, Wd2, bg1.reshape(1, D), bd2.reshape(1, D))


def _topk_body(pc, pct, wbp, bbp, knn_out, gidx_out):
    b = pl.program_id(0)
    pcb = pc[0]
    pctb = pct[0]
    cxr = pcb[:, 0:1]
    cyr = pcb[:, 1:2]
    hwr = 0.5 * pcb[:, 2:3]
    hhr = 0.5 * pcb[:, 3:4]
    x1r, y1r = cxr - hwr, cyr - hhr
    x2r, y2r = cxr + hwr, cyr + hhr
    cxc = pctb[0:1, :]
    cyc = pctb[1:2, :]
    hwc = 0.5 * pctb[2:3, :]
    hhc = 0.5 * pctb[3:4, :]
    x1c, y1c = cxc - hwc, cyc - hhc
    x2c, y2c = cxc + hwc, cyc + hhc

    dx = cxr - cxc
    dy = cyr - cyc
    dis = jnp.sqrt(dx * dx + dy * dy)
    ow = jnp.clip(jnp.minimum(x2r, x2c) - jnp.maximum(x1r, x1c), 0.0, None)
    oh = jnp.clip(jnp.minimum(y2r, y2c) - jnp.maximum(y1r, y1c), 0.0, None)
    uw = jnp.clip(jnp.maximum(x2r, x2c) - jnp.minimum(x1r, x1c), 0.0, None)
    uh = jnp.clip(jnp.maximum(y2r, y2c) - jnp.minimum(y1r, y1c), 0.0, None)
    iou = (ow * oh) / (uw * uh + 1e-06)
    s = jnp.concatenate([dis, iou], axis=0)
    ri = lax.broadcasted_iota(jnp.int32, (RB, 2 * RB), 0)
    ci = lax.broadcasted_iota(jnp.int32, (RB, 2 * RB), 1)
    wc = jnp.where(ci == ri, wbp[0], 0.0) + jnp.where(ci == ri + RB, wbp[1], 0.0)
    vals = jnp.dot(wc, s, preferred_element_type=jnp.float32) + bbp[0]

    lane = lax.broadcasted_iota(jnp.int32, (RB, NP), 1)
    inf = jnp.float32(np.inf)
    vals = jnp.where(lane >= N, inf, vals)
    cols = []
    for _ in range(K):
        m = jnp.min(vals, axis=1, keepdims=True)
        cand = jnp.where(vals == m, lane, jnp.int32(1 << 30))
        j = jnp.min(cand, axis=1, keepdims=True)
        vals = jnp.where(lane == j, inf, vals)
        cols.append(j)
    knn = jnp.concatenate(cols, axis=1)
    knn_out[0] = knn
    gidx_out[0] = knn + b * NP


def _topk(pc_pad, pct_pad, wbp, bbp):
    grid = (B, NP // RB)
    return pl.pallas_call(
        _topk_body,
        grid=grid,
        in_specs=[
            pl.BlockSpec((1, RB, 4), lambda b, r: (b, r, 0)),
            pl.BlockSpec((1, 8, NP), lambda b, r: (b, 0, 0)),
            pl.BlockSpec(memory_space=pltpu.SMEM),
            pl.BlockSpec(memory_space=pltpu.SMEM),
        ],
        out_specs=(
            pl.BlockSpec((1, RB, K), lambda b, r: (b, r, 0)),
            pl.BlockSpec((1, RB, K), lambda b, r: (b, r, 0)),
        ),
        out_shape=(
            jax.ShapeDtypeStruct((B, N, K), jnp.int32),
            jax.ShapeDtypeStruct((B, NP, K), jnp.int32),
        ),
    )(pc_pad, pct_pad, wbp, bbp)


def _pernode_body(feat, pos, w1, b1, wd1, wv, wkg, wqg, cb, t_out, p_out):
    x = jnp.dot(feat[0], w1[...], preferred_element_type=jnp.float32) + b1[...]
    pd = jnp.dot(pos[0], wd1[...], preferred_element_type=jnp.float32)
    d = jnp.dot(x, wkg[...], preferred_element_type=jnp.float32)
    v = jnp.dot(x, wv[...], preferred_element_type=jnp.float32)
    c = jnp.dot(x, wqg[...], preferred_element_type=jnp.float32) + cb[...]
    t = jnp.concatenate([pd, d, v], axis=1)
    tl = t[:, :HW].astype(jnp.bfloat16).astype(jnp.float32)
    th = t[:, HW:].astype(jnp.bfloat16).astype(jnp.float32)
    word = (lax.bitcast_convert_type(th, jnp.uint32)
            | (lax.bitcast_convert_type(tl, jnp.uint32) >> 16))
    t_out[0] = lax.bitcast_convert_type(word, jnp.int32)
    p_out[0] = jnp.concatenate([pd, c], axis=1)


def _pernode(feat_pad, pos_pad, W1, b1, Wd1, Wv, Wkg, Wqg, cb):
    grid = (B, NP // RB)
    full = lambda b, r: (0, 0)
    return pl.pallas_call(
        _pernode_body,
        grid=grid,
        in_specs=[
            pl.BlockSpec((1, RB, D), lambda b, r: (b, r, 0)),
            pl.BlockSpec((1, RB, D), lambda b, r: (b, r, 0)),
            pl.BlockSpec((D, D), full),
            pl.BlockSpec((1, D), full),
            pl.BlockSpec((D, D), full),
            pl.BlockSpec((D, D), full),
            pl.BlockSpec((D, D), full),
            pl.BlockSpec((D, D), full),
            pl.BlockSpec((1, D), full),
        ],
        out_specs=(
            pl.BlockSpec((1, RB, HW), lambda b, r: (b, r, 0)),
            pl.BlockSpec((1, RB, 2 * D), lambda b, r: (b, r, 0)),
        ),
        out_shape=(
            jax.ShapeDtypeStruct((B, NP, HW), jnp.int32),
            jax.ShapeDtypeStruct((B, NP, 2 * D), jnp.float32),
        ),
    )(feat_pad, pos_pad, W1, b1.reshape(1, D), Wd1, Wv, Wkg, Wqg, cb)


def _sc_gather_body(table_hbm, idx_hbm, out_hbm,
                    idx0, idx1, rows0, rows1, si0, si1, sg, sw0, sw1):
    wid = lax.axis_index("s") * SC_NC + lax.axis_index("c")
    base = wid * ROWS_PER_W
    nch = ROWS_PER_W // CHUNK
    idx_v = (idx0, idx1)
    rows_v = (rows0, rows1)
    si = (si0, si1)
    sw = (sw0, sw1)

    pltpu.async_copy(idx_hbm.at[pl.ds(base, CHUNK)], idx_v[0], si[0])
    for j in range(nch):
        p = j % 2
        si_c = pltpu.make_async_copy(
            idx_hbm.at[pl.ds(base + j * CHUNK, CHUNK)], idx_v[p], si[p])
        si_c.wait()
        if j + 1 < nch:
            pltpu.async_copy(idx_hbm.at[pl.ds(base + (j + 1) * CHUNK, CHUNK)],
                             idx_v[(j + 1) % 2], si[(j + 1) % 2])
        if j >= 2:
            pltpu.make_async_copy(
                rows_v[p], out_hbm.at[pl.ds(base + (j - 2) * CHUNK, CHUNK)],
                sw[p]).wait()
        pltpu.async_copy(table_hbm.at[idx_v[p]], rows_v[p], sg).wait()
        pltpu.async_copy(rows_v[p], out_hbm.at[pl.ds(base + j * CHUNK, CHUNK)],
                         sw[p])
    for j in range(max(nch - 2, 0), nch):
        p = j % 2
        pltpu.make_async_copy(
            rows_v[p], out_hbm.at[pl.ds(base + j * CHUNK, CHUNK)], sw[p]).wait()


def _sc_gather(table_flat, gidx_flat):
    mesh = plsc.VectorSubcoreMesh(core_axis_name="c", subcore_axis_name="s")
    kfn = functools.partial(
        pl.kernel,
        mesh=mesh,
        out_type=jax.ShapeDtypeStruct((BROWS, HW), jnp.int32),
        scratch_types=[
            pltpu.VMEM((CHUNK,), jnp.int32),
            pltpu.VMEM((CHUNK,), jnp.int32),
            pltpu.VMEM((CHUNK, HW), jnp.int32),
            pltpu.VMEM((CHUNK, HW), jnp.int32),
            pltpu.SemaphoreType.DMA,
            pltpu.SemaphoreType.DMA,
            pltpu.SemaphoreType.DMA,
            pltpu.SemaphoreType.DMA,
            pltpu.SemaphoreType.DMA,
        ],
    )(_sc_gather_body)
    return kfn(table_flat, gidx_flat)


def _main_body(g, p, feat, wcat, wg2, w2, bd1, bd2, bg2, b2, out):
    gw = lax.bitcast_convert_type(g[...], jnp.uint32)
    lo = lax.bitcast_convert_type(gw << 16, jnp.float32)
    hi = lax.bitcast_convert_type(gw & jnp.uint32(0xFFFF0000), jnp.float32)
    pdg = lo[:, 0:D].reshape(Q, K, D)
    dg = jnp.concatenate([lo[:, D:HW], hi[:, 0:HW - D]], axis=1).reshape(Q, K, D)
    vg = hi[:, HW - D:HW].reshape(Q, K, D)
    pb = p[...]
    pdq = pb[:, 0:D]
    cq = pb[:, D:2 * D]

    h = pdq[:, None, :] - pdg + bd1[...]
    r = jnp.maximum(h, 0.0).reshape(Q * K, D).astype(jnp.bfloat16)
    pa = jnp.dot(r, wcat[...], preferred_element_type=jnp.float32)
    pe = pa[:, 0:D].reshape(Q, K, D) + bd2[...]
    a1 = pa[:, D:2 * D].reshape(Q, K, D) + cq[:, None, :] - dg
    a1 = jnp.maximum(a1, 0.0).reshape(Q * K, D).astype(jnp.bfloat16)
    a2 = jnp.dot(a1, wg2[...], preferred_element_type=jnp.float32)
    a2 = a2.reshape(Q, K, D) + bg2[...]

    z = a2 * jnp.float32(1.0 / 16.0)
    zmax = jnp.max(z, axis=1, keepdims=True)
    e = jnp.exp(z - zmax)
    attn = e / jnp.sum(e, axis=1, keepdims=True)
    o = jnp.sum(attn * (vg + pe), axis=1)
    res = jnp.dot(o, w2[...], preferred_element_type=jnp.float32)
    out[...] = res + b2[...] + feat[...]


def _main(g_rows, p_arr, feat_b, Wcat, Wg2, W2, bd1, bd2, bg2, b2):
    grid = (NP // Q,)
    full = lambda q: (0, 0)
    return pl.pallas_call(
        _main_body,
        grid=grid,
        in_specs=[
            pl.BlockSpec((Q * K, HW), lambda q: (q, 0)),
            pl.BlockSpec((Q, 2 * D), lambda q: (q, 0)),
            pl.BlockSpec((Q, D), lambda q: (q, 0)),
            pl.BlockSpec((D, 2 * D), full),
            pl.BlockSpec((D, D), full),
            pl.BlockSpec((D, D), full),
            pl.BlockSpec((1, D), full),
            pl.BlockSpec((1, D), full),
            pl.BlockSpec((1, D), full),
            pl.BlockSpec((1, D), full),
        ],
        out_specs=pl.BlockSpec((Q, D), lambda q: (q, 0)),
        out_shape=jax.ShapeDtypeStruct((NP, D), jnp.float32),
    )(g_rows, p_arr, feat_b, Wcat, Wg2, W2,
      bd1.reshape(1, D), bd2.reshape(1, D), bg2.reshape(1, D),
      b2.reshape(1, D))


def kernel(features, pos, pos_center, W1, b1, W2, b2, Wg1, bg1, Wg2, bg2,
           Wd1, bd1, Wd2, bd2, Wbp, bbp, Wq, Wk, Wv):
    pad_n = [(0, 0), (0, NP - N), (0, 0)]
    feat_pad = jnp.pad(features, pad_n)
    pos_pad = jnp.pad(pos, pad_n)
    pc_pad = jnp.pad(pos_center, pad_n)
    pct_pad = jnp.pad(pos_center.transpose(0, 2, 1), [(0, 0), (0, 4), (0, NP - N)])

    Wkg, Wqg, M, cb = _prep(Wk, Wg1, Wq, Wd2, bg1, bd2)
    knn_idx, gidx = _topk(pc_pad, pct_pad, Wbp.reshape(2), bbp)
    T, P = _pernode(feat_pad, pos_pad, W1, b1, Wd1, Wv, Wkg, Wqg, cb)

    T2 = T.reshape(B * NP, HW)
    Wcat = jnp.concatenate([Wd2, M], axis=1).astype(jnp.bfloat16)
    Wg2b = Wg2.astype(jnp.bfloat16)
    res = []
    for b in range(B):
        g_b = _sc_gather(T2, gidx[b].reshape(BROWS))
        res.append(_main(g_b, P[b], feat_pad[b], Wcat, Wg2b, W2,
                         bd1, bd2, bg2, b2))
    return jnp.stack(res, axis=0)[:, :N, :], knn_idx


# speedup vs baseline: 10.5919x; 1.0870x over previous
import functools

import jax
import jax.numpy as jnp
import numpy as np
from jax import lax
from jax.experimental import pallas as pl
from jax.experimental.pallas import tpu as pltpu
from jax.experimental.pallas import tpu_sc as plsc

B, N, D, K = 4, 1000, 256, 16
NP = 1024
RB = 256
Q = 128

SC_NC, SC_NS = 2, 16
SC_NW = SC_NC * SC_NS
GROWS = B * NP * K
HW = 3 * D // 2
BROWS = NP * K
ROWS_PER_W = BROWS // SC_NW
CHUNK = 128


def _prep_body(wk, wg1, wq, wd2, bg1, bd2, wkg, wqg, m, cb):
    g1 = wg1[...]
    wkg[...] = jnp.dot(wk[...], g1, preferred_element_type=jnp.float32)
    wqg[...] = jnp.dot(wq[...], g1, preferred_element_type=jnp.float32)
    m[...] = jnp.dot(wd2[...], g1, preferred_element_type=jnp.float32)
    cb[...] = bg1[...] + jnp.dot(bd2[...], g1,
                                 preferred_element_type=jnp.float32)


def _prep(Wk, Wg1, Wq, Wd2, bg1, bd2):
    f = jax.ShapeDtypeStruct((D, D), jnp.float32)
    r = jax.ShapeDtypeStruct((1, D), jnp.float32)
    return pl.pallas_call(
        _prep_body,
        out_shape=(f, f, f, r),
    )(Wk, Wg1, Wq, Wd2, bg1.reshape(1, D), bd2.reshape(1, D))


def _topk_body(pc, pct, wbp, bbp, boff, knn_out, gidx_out):
    pcb = pc[...]
    pctb = pct[...]
    cxr = pcb[:, 0:1]
    cyr = pcb[:, 1:2]
    hwr = 0.5 * pcb[:, 2:3]
    hhr = 0.5 * pcb[:, 3:4]
    x1r, y1r = cxr - hwr, cyr - hhr
    x2r, y2r = cxr + hwr, cyr + hhr
    cxc = pctb[0:1, :]
    cyc = pctb[1:2, :]
    hwc = 0.5 * pctb[2:3, :]
    hhc = 0.5 * pctb[3:4, :]
    x1c, y1c = cxc - hwc, cyc - hhc
    x2c, y2c = cxc + hwc, cyc + hhc

    dx = cxr - cxc
    dy = cyr - cyc
    dis = jnp.sqrt(dx * dx + dy * dy)
    ow = jnp.clip(jnp.minimum(x2r, x2c) - jnp.maximum(x1r, x1c), 0.0, None)
    oh = jnp.clip(jnp.minimum(y2r, y2c) - jnp.maximum(y1r, y1c), 0.0, None)
    uw = jnp.clip(jnp.maximum(x2r, x2c) - jnp.minimum(x1r, x1c), 0.0, None)
    uh = jnp.clip(jnp.maximum(y2r, y2c) - jnp.minimum(y1r, y1c), 0.0, None)
    iou = (ow * oh) / (uw * uh + 1e-06)
    s = jnp.concatenate([dis, iou], axis=0)
    ri = lax.broadcasted_iota(jnp.int32, (RB, 2 * RB), 0)
    ci = lax.broadcasted_iota(jnp.int32, (RB, 2 * RB), 1)
    wc = jnp.where(ci == ri, wbp[0], 0.0) + jnp.where(ci == ri + RB, wbp[1], 0.0)
    vals = jnp.dot(wc, s, preferred_element_type=jnp.float32) + bbp[0]

    lane = lax.broadcasted_iota(jnp.int32, (RB, NP), 1)
    inf = jnp.float32(np.inf)
    vals = jnp.where(lane >= N, inf, vals)
    cols = []
    for _ in range(K):
        m = jnp.min(vals, axis=1, keepdims=True)
        cand = jnp.where(vals == m, lane, jnp.int32(1 << 30))
        j = jnp.min(cand, axis=1, keepdims=True)
        vals = jnp.where(lane == j, inf, vals)
        cols.append(j)
    knn = jnp.concatenate(cols, axis=1)
    knn_out[...] = knn
    gidx_out[...] = knn + boff[0]


def _topk(pc_b, pct_b, wbp, bbp, boff):
    grid = (NP // RB,)
    return pl.pallas_call(
        _topk_body,
        grid=grid,
        in_specs=[
            pl.BlockSpec((RB, 4), lambda r: (r, 0)),
            pl.BlockSpec((8, NP), lambda r: (0, 0)),
            pl.BlockSpec(memory_space=pltpu.SMEM),
            pl.BlockSpec(memory_space=pltpu.SMEM),
            pl.BlockSpec(memory_space=pltpu.SMEM),
        ],
        out_specs=(
            pl.BlockSpec((RB, K), lambda r: (r, 0)),
            pl.BlockSpec((RB, K), lambda r: (r, 0)),
        ),
        out_shape=(
            jax.ShapeDtypeStruct((N, K), jnp.int32),
            jax.ShapeDtypeStruct((NP, K), jnp.int32),
        ),
    )(pc_b, pct_b, wbp, bbp, boff)


def _pernode_body(feat, pos, w1, b1, wd1, wv, wkg, wqg, cb, t_out, p_out):
    x = jnp.dot(feat[0], w1[...], preferred_element_type=jnp.float32) + b1[...]
    pd = jnp.dot(pos[0], wd1[...], preferred_element_type=jnp.float32)
    d = jnp.dot(x, wkg[...], preferred_element_type=jnp.float32)
    v = jnp.dot(x, wv[...], preferred_element_type=jnp.float32)
    c = jnp.dot(x, wqg[...], preferred_element_type=jnp.float32) + cb[...]
    t = jnp.concatenate([pd, d, v], axis=1)
    tl = t[:, :HW].astype(jnp.bfloat16).astype(jnp.float32)
    th = t[:, HW:].astype(jnp.bfloat16).astype(jnp.float32)
    word = (lax.bitcast_convert_type(th, jnp.uint32)
            | (lax.bitcast_convert_type(tl, jnp.uint32) >> 16))
    t_out[0] = lax.bitcast_convert_type(word, jnp.int32)
    p_out[0] = jnp.concatenate([pd, c], axis=1)


def _pernode(feat_pad, pos_pad, W1, b1, Wd1, Wv, Wkg, Wqg, cb):
    grid = (B, NP // RB)
    full = lambda b, r: (0, 0)
    return pl.pallas_call(
        _pernode_body,
        grid=grid,
        in_specs=[
            pl.BlockSpec((1, RB, D), lambda b, r: (b, r, 0)),
            pl.BlockSpec((1, RB, D), lambda b, r: (b, r, 0)),
            pl.BlockSpec((D, D), full),
            pl.BlockSpec((1, D), full),
            pl.BlockSpec((D, D), full),
            pl.BlockSpec((D, D), full),
            pl.BlockSpec((D, D), full),
            pl.BlockSpec((D, D), full),
            pl.BlockSpec((1, D), full),
        ],
        out_specs=(
            pl.BlockSpec((1, RB, HW), lambda b, r: (b, r, 0)),
            pl.BlockSpec((1, RB, 2 * D), lambda b, r: (b, r, 0)),
        ),
        out_shape=(
            jax.ShapeDtypeStruct((B, NP, HW), jnp.int32),
            jax.ShapeDtypeStruct((B, NP, 2 * D), jnp.float32),
        ),
    )(feat_pad, pos_pad, W1, b1.reshape(1, D), Wd1, Wv, Wkg, Wqg, cb)


def _sc_gather_body(table_hbm, idx_hbm, out_hbm,
                    idx0, idx1, rows0, rows1, si0, si1, sg, sw0, sw1):
    wid = lax.axis_index("s") * SC_NC + lax.axis_index("c")
    base = wid * ROWS_PER_W
    nch = ROWS_PER_W // CHUNK
    idx_v = (idx0, idx1)
    rows_v = (rows0, rows1)
    si = (si0, si1)
    sw = (sw0, sw1)

    pltpu.async_copy(idx_hbm.at[pl.ds(base, CHUNK)], idx_v[0], si[0])
    for j in range(nch):
        p = j % 2
        si_c = pltpu.make_async_copy(
            idx_hbm.at[pl.ds(base + j * CHUNK, CHUNK)], idx_v[p], si[p])
        si_c.wait()
        if j + 1 < nch:
            pltpu.async_copy(idx_hbm.at[pl.ds(base + (j + 1) * CHUNK, CHUNK)],
                             idx_v[(j + 1) % 2], si[(j + 1) % 2])
        if j >= 2:
            pltpu.make_async_copy(
                rows_v[p], out_hbm.at[pl.ds(base + (j - 2) * CHUNK, CHUNK)],
                sw[p]).wait()
        pltpu.async_copy(table_hbm.at[idx_v[p]], rows_v[p], sg).wait()
        pltpu.async_copy(rows_v[p], out_hbm.at[pl.ds(base + j * CHUNK, CHUNK)],
                         sw[p])
    for j in range(max(nch - 2, 0), nch):
        p = j % 2
        pltpu.make_async_copy(
            rows_v[p], out_hbm.at[pl.ds(base + j * CHUNK, CHUNK)], sw[p]).wait()


def _sc_gather(table_flat, gidx_flat):
    mesh = plsc.VectorSubcoreMesh(core_axis_name="c", subcore_axis_name="s")
    kfn = functools.partial(
        pl.kernel,
        mesh=mesh,
        out_type=jax.ShapeDtypeStruct((BROWS, HW), jnp.int32),
        scratch_types=[
            pltpu.VMEM((CHUNK,), jnp.int32),
            pltpu.VMEM((CHUNK,), jnp.int32),
            pltpu.VMEM((CHUNK, HW), jnp.int32),
            pltpu.VMEM((CHUNK, HW), jnp.int32),
            pltpu.SemaphoreType.DMA,
            pltpu.SemaphoreType.DMA,
            pltpu.SemaphoreType.DMA,
            pltpu.SemaphoreType.DMA,
            pltpu.SemaphoreType.DMA,
        ],
    )(_sc_gather_body)
    return kfn(table_flat, gidx_flat)


def _main_body(g, p, feat, wcat, wg2, w2, bd1, bd2, bg2, b2, out):
    gw = lax.bitcast_convert_type(g[...], jnp.uint32)
    lo = lax.bitcast_convert_type(gw << 16, jnp.float32)
    hi = lax.bitcast_convert_type(gw & jnp.uint32(0xFFFF0000), jnp.float32)
    pdg = lo[:, 0:D].reshape(Q, K, D)
    dg = jnp.concatenate([lo[:, D:HW], hi[:, 0:HW - D]], axis=1).reshape(Q, K, D)
    vg = hi[:, HW - D:HW].reshape(Q, K, D)
    pb = p[...]
    pdq = pb[:, 0:D]
    cq = pb[:, D:2 * D]

    h = pdq[:, None, :] - pdg + bd1[...]
    r = jnp.maximum(h, 0.0).reshape(Q * K, D).astype(jnp.bfloat16)
    pa = jnp.dot(r, wcat[...], preferred_element_type=jnp.float32)
    pe = pa[:, 0:D].reshape(Q, K, D) + bd2[...]
    a1 = pa[:, D:2 * D].reshape(Q, K, D) + cq[:, None, :] - dg
    a1 = jnp.maximum(a1, 0.0).reshape(Q * K, D).astype(jnp.bfloat16)
    a2 = jnp.dot(a1, wg2[...], preferred_element_type=jnp.float32)
    a2 = a2.reshape(Q, K, D) + bg2[...]

    z = a2 * jnp.float32(1.0 / 16.0)
    zmax = jnp.max(z, axis=1, keepdims=True)
    e = jnp.exp(z - zmax)
    attn = e / jnp.sum(e, axis=1, keepdims=True)
    o = jnp.sum(attn * (vg + pe), axis=1)
    res = jnp.dot(o, w2[...], preferred_element_type=jnp.float32)
    out[...] = res + b2[...] + feat[...]


def _main(g_rows, p_arr, feat_b, Wcat, Wg2, W2, bd1, bd2, bg2, b2):
    grid = (NP // Q,)
    full = lambda q: (0, 0)
    return pl.pallas_call(
        _main_body,
        grid=grid,
        in_specs=[
            pl.BlockSpec((Q * K, HW), lambda q: (q, 0)),
            pl.BlockSpec((Q, 2 * D), lambda q: (q, 0)),
            pl.BlockSpec((Q, D), lambda q: (q, 0)),
            pl.BlockSpec((D, 2 * D), full),
            pl.BlockSpec((D, D), full),
            pl.BlockSpec((D, D), full),
            pl.BlockSpec((1, D), full),
            pl.BlockSpec((1, D), full),
            pl.BlockSpec((1, D), full),
            pl.BlockSpec((1, D), full),
        ],
        out_specs=pl.BlockSpec((Q, D), lambda q: (q, 0)),
        out_shape=jax.ShapeDtypeStruct((NP, D), jnp.float32),
    )(g_rows, p_arr, feat_b, Wcat, Wg2, W2,
      bd1.reshape(1, D), bd2.reshape(1, D), bg2.reshape(1, D),
      b2.reshape(1, D))


def kernel(features, pos, pos_center, W1, b1, W2, b2, Wg1, bg1, Wg2, bg2,
           Wd1, bd1, Wd2, bd2, Wbp, bbp, Wq, Wk, Wv):
    pad_n = [(0, 0), (0, NP - N), (0, 0)]
    feat_pad = jnp.pad(features, pad_n)
    pos_pad = jnp.pad(pos, pad_n)
    pc_pad = jnp.pad(pos_center, pad_n)
    pct_pad = jnp.pad(pos_center.transpose(0, 2, 1), [(0, 0), (0, 4), (0, NP - N)])

    Wkg, Wqg, M, cb = _prep(Wk, Wg1, Wq, Wd2, bg1, bd2)
    T, P = _pernode(feat_pad, pos_pad, W1, b1, Wd1, Wv, Wkg, Wqg, cb)
    wbp2 = Wbp.reshape(2)
    knns, gidxs = [], []
    for b in range(B):
        knn_b, gidx_b = _topk(pc_pad[b], pct_pad[b], wbp2, bbp,
                              jnp.full((1,), b * NP, jnp.int32))
        knns.append(knn_b)
        gidxs.append(gidx_b)

    T2 = T.reshape(B * NP, HW)
    Wcat = jnp.concatenate([Wd2, M], axis=1).astype(jnp.bfloat16)
    Wg2b = Wg2.astype(jnp.bfloat16)
    res = []
    for b in range(B):
        g_b = _sc_gather(T2, gidxs[b].reshape(BROWS))
        res.append(_main(g_b, P[b], feat_pad[b], Wcat, Wg2b, W2,
                         bd1, bd2, bg2, b2))
    return jnp.stack(res, axis=0)[:, :N, :], jnp.stack(knns, axis=0)


# speedup vs baseline: 12.0203x; 1.1349x over previous
import functools

import jax
import jax.numpy as jnp
import numpy as np
from jax import lax
from jax.experimental import pallas as pl
from jax.experimental.pallas import tpu as pltpu
from jax.experimental.pallas import tpu_sc as plsc

B, N, D, K = 4, 1000, 256, 16
NP = 1024
RB = 512
Q = 256

SC_NC, SC_NS = 2, 16
SC_NW = SC_NC * SC_NS
GROWS = B * NP * K
HW = 3 * D // 2
BROWS = NP * K
ROWS_PER_W = BROWS // SC_NW
CHUNK = 128
IR = CHUNK // K


def _prep_body(wk, wg1, wq, wd2, wg2, bg1, bd2, wkg, wqg, wcat, wg2b, cb):
    g1 = wg1[...]
    wkg[...] = jnp.dot(wk[...], g1, preferred_element_type=jnp.float32)
    wqg[...] = jnp.dot(wq[...], g1, preferred_element_type=jnp.float32)
    m = jnp.dot(wd2[...], g1, preferred_element_type=jnp.float32)
    wcat[...] = jnp.concatenate([wd2[...], m], axis=1).astype(jnp.bfloat16)
    wg2b[...] = wg2[...].astype(jnp.bfloat16)
    cb[...] = bg1[...] + jnp.dot(bd2[...], g1,
                                 preferred_element_type=jnp.float32)


def _prep(Wk, Wg1, Wq, Wd2, Wg2, bg1, bd2):
    f = jax.ShapeDtypeStruct((D, D), jnp.float32)
    return pl.pallas_call(
        _prep_body,
        out_shape=(f, f,
                   jax.ShapeDtypeStruct((D, 2 * D), jnp.bfloat16),
                   jax.ShapeDtypeStruct((D, D), jnp.bfloat16),
                   jax.ShapeDtypeStruct((1, D), jnp.float32)),
    )(Wk, Wg1, Wq, Wd2, Wg2, bg1.reshape(1, D), bd2.reshape(1, D))


def _topk_body(pc, pct, wbp, bbp, knn_out, gidx_out):
    pcb = pc[...]
    pctb = pct[...]
    cxr = pcb[:, 0:1]
    cyr = pcb[:, 1:2]
    hwr = 0.5 * pcb[:, 2:3]
    hhr = 0.5 * pcb[:, 3:4]
    x1r, y1r = cxr - hwr, cyr - hhr
    x2r, y2r = cxr + hwr, cyr + hhr
    cxc = pctb[0:1, :]
    cyc = pctb[1:2, :]
    hwc = 0.5 * pctb[2:3, :]
    hhc = 0.5 * pctb[3:4, :]
    x1c, y1c = cxc - hwc, cyc - hhc
    x2c, y2c = cxc + hwc, cyc + hhc

    dx = cxr - cxc
    dy = cyr - cyc
    dis = jnp.sqrt(dx * dx + dy * dy)
    ow = jnp.clip(jnp.minimum(x2r, x2c) - jnp.maximum(x1r, x1c), 0.0, None)
    oh = jnp.clip(jnp.minimum(y2r, y2c) - jnp.maximum(y1r, y1c), 0.0, None)
    uw = jnp.clip(jnp.maximum(x2r, x2c) - jnp.minimum(x1r, x1c), 0.0, None)
    uh = jnp.clip(jnp.maximum(y2r, y2c) - jnp.minimum(y1r, y1c), 0.0, None)
    iou = (ow * oh) / (uw * uh + 1e-06)
    s = jnp.concatenate([dis, iou], axis=0)
    ri = lax.broadcasted_iota(jnp.int32, (RB, 2 * RB), 0)
    ci = lax.broadcasted_iota(jnp.int32, (RB, 2 * RB), 1)
    wc = jnp.where(ci == ri, wbp[0], 0.0) + jnp.where(ci == ri + RB, wbp[1], 0.0)
    vals = jnp.dot(wc, s, preferred_element_type=jnp.float32) + bbp[0]

    lane = lax.broadcasted_iota(jnp.int32, (RB, NP), 1)
    inf = jnp.float32(np.inf)
    vals = jnp.where(lane >= N, inf, vals)
    cols = []
    for _ in range(K):
        m = jnp.min(vals, axis=1, keepdims=True)
        cand = jnp.where(vals == m, lane, jnp.int32(1 << 30))
        j = jnp.min(cand, axis=1, keepdims=True)
        vals = jnp.where(lane == j, inf, vals)
        cols.append(j)
    knn = jnp.concatenate(cols, axis=1)
    knn_out[...] = knn
    gidx_out[...] = knn


def _topk(pc_b, pct_b, wbp, bbp):
    grid = (NP // RB,)
    return pl.pallas_call(
        _topk_body,
        grid=grid,
        in_specs=[
            pl.BlockSpec((RB, 4), lambda r: (r, 0)),
            pl.BlockSpec((8, NP), lambda r: (0, 0)),
            pl.BlockSpec(memory_space=pltpu.SMEM),
            pl.BlockSpec(memory_space=pltpu.SMEM),
        ],
        out_specs=(
            pl.BlockSpec((RB, K), lambda r: (r, 0)),
            pl.BlockSpec((RB, K), lambda r: (r, 0)),
        ),
        out_shape=(
            jax.ShapeDtypeStruct((N, K), jnp.int32),
            jax.ShapeDtypeStruct((NP, K), jnp.int32),
        ),
    )(pc_b, pct_b, wbp, bbp)


def _pernode_body(feat, pos, w1, b1, wd1, wv, wkg, wqg, cb, t_out, p_out):
    x = jnp.dot(feat[0], w1[...], preferred_element_type=jnp.float32) + b1[...]
    pd = jnp.dot(pos[0], wd1[...], preferred_element_type=jnp.float32)
    d = jnp.dot(x, wkg[...], preferred_element_type=jnp.float32)
    v = jnp.dot(x, wv[...], preferred_element_type=jnp.float32)
    c = jnp.dot(x, wqg[...], preferred_element_type=jnp.float32) + cb[...]
    t = jnp.concatenate([pd, d, v], axis=1)
    tl = t[:, :HW].astype(jnp.bfloat16).astype(jnp.float32)
    th = t[:, HW:].astype(jnp.bfloat16).astype(jnp.float32)
    word = (lax.bitcast_convert_type(th, jnp.uint32)
            | (lax.bitcast_convert_type(tl, jnp.uint32) >> 16))
    t_out[0] = lax.bitcast_convert_type(word, jnp.int32)
    p_out[0] = jnp.concatenate([pd, c], axis=1)


def _pernode(features, pos, W1, b1, Wd1, Wv, Wkg, Wqg, cb):
    grid = (B, NP // RB)
    full = lambda b, r: (0, 0)
    return pl.pallas_call(
        _pernode_body,
        grid=grid,
        in_specs=[
            pl.BlockSpec((1, RB, D), lambda b, r: (b, r, 0)),
            pl.BlockSpec((1, RB, D), lambda b, r: (b, r, 0)),
            pl.BlockSpec((D, D), full),
            pl.BlockSpec((1, D), full),
            pl.BlockSpec((D, D), full),
            pl.BlockSpec((D, D), full),
            pl.BlockSpec((D, D), full),
            pl.BlockSpec((D, D), full),
            pl.BlockSpec((1, D), full),
        ],
        out_specs=(
            pl.BlockSpec((1, RB, HW), lambda b, r: (b, r, 0)),
            pl.BlockSpec((1, RB, 2 * D), lambda b, r: (b, r, 0)),
        ),
        out_shape=(
            jax.ShapeDtypeStruct((B, N, HW), jnp.int32),
            jax.ShapeDtypeStruct((B, N, 2 * D), jnp.float32),
        ),
    )(features, pos, W1, b1.reshape(1, D), Wd1, Wv, Wkg, Wqg, cb)


def _sc_gather_body(table_hbm, idx_hbm, out_hbm,
                    idx2d0, idx2d1, idx0, idx1, rows0, rows1,
                    si0, si1, sg, sw0, sw1):
    wid = lax.axis_index("s") * SC_NC + lax.axis_index("c")
    base = wid * ROWS_PER_W
    rbase = wid * (ROWS_PER_W // K)
    nch = ROWS_PER_W // CHUNK
    idx2d = (idx2d0, idx2d1)
    idx_v = (idx0, idx1)
    rows_v = (rows0, rows1)
    si = (si0, si1)
    sw = (sw0, sw1)

    def flatten(p):
        for r in range(IR):
            idx_v[p][pl.ds(r * K, K)] = idx2d[p][r]

    pltpu.async_copy(idx_hbm.at[pl.ds(rbase, IR)], idx2d[0], si[0])
    for j in range(nch):
        p = j % 2
        si_c = pltpu.make_async_copy(
            idx_hbm.at[pl.ds(rbase + j * IR, IR)], idx2d[p], si[p])
        si_c.wait()
        if j + 1 < nch:
            pltpu.async_copy(idx_hbm.at[pl.ds(rbase + (j + 1) * IR, IR)],
                             idx2d[(j + 1) % 2], si[(j + 1) % 2])
        flatten(p)
        if j >= 2:
            pltpu.make_async_copy(
                rows_v[p], out_hbm.at[pl.ds(base + (j - 2) * CHUNK, CHUNK)],
                sw[p]).wait()
        pltpu.async_copy(table_hbm.at[idx_v[p]], rows_v[p], sg).wait()
        pltpu.async_copy(rows_v[p], out_hbm.at[pl.ds(base + j * CHUNK, CHUNK)],
                         sw[p])
    for j in range(max(nch - 2, 0), nch):
        p = j % 2
        pltpu.make_async_copy(
            rows_v[p], out_hbm.at[pl.ds(base + j * CHUNK, CHUNK)], sw[p]).wait()


def _sc_gather(table_b, gidx2d):
    mesh = plsc.VectorSubcoreMesh(core_axis_name="c", subcore_axis_name="s")
    kfn = functools.partial(
        pl.kernel,
        mesh=mesh,
        out_type=jax.ShapeDtypeStruct((BROWS, HW), jnp.int32),
        scratch_types=[
            pltpu.VMEM((IR, K), jnp.int32),
            pltpu.VMEM((IR, K), jnp.int32),
            pltpu.VMEM((CHUNK,), jnp.int32),
            pltpu.VMEM((CHUNK,), jnp.int32),
            pltpu.VMEM((CHUNK, HW), jnp.int32),
            pltpu.VMEM((CHUNK, HW), jnp.int32),
            pltpu.SemaphoreType.DMA,
            pltpu.SemaphoreType.DMA,
            pltpu.SemaphoreType.DMA,
            pltpu.SemaphoreType.DMA,
            pltpu.SemaphoreType.DMA,
        ],
    )(_sc_gather_body)
    return kfn(table_b, gidx2d)


def _main_body(g, p, feat, wcat, wg2, w2, bd1, bd2, bg2, b2, out):
    gw = lax.bitcast_convert_type(g[...], jnp.uint32)
    lo = lax.bitcast_convert_type(gw << 16, jnp.float32)
    hi = lax.bitcast_convert_type(gw & jnp.uint32(0xFFFF0000), jnp.float32)
    pdg = lo[:, 0:D].reshape(Q, K, D)
    dg = jnp.concatenate([lo[:, D:HW], hi[:, 0:HW - D]], axis=1).reshape(Q, K, D)
    vg = hi[:, HW - D:HW].reshape(Q, K, D)
    pb = p[...]
    pdq = pb[:, 0:D]
    cq = pb[:, D:2 * D]

    h = pdq[:, None, :] - pdg + bd1[...]
    r = jnp.maximum(h, 0.0).reshape(Q * K, D).astype(jnp.bfloat16)
    pa = jnp.dot(r, wcat[...], preferred_element_type=jnp.float32)
    pe = pa[:, 0:D].reshape(Q, K, D) + bd2[...]
    a1 = pa[:, D:2 * D].reshape(Q, K, D) + cq[:, None, :] - dg
    a1 = jnp.maximum(a1, 0.0).reshape(Q * K, D).astype(jnp.bfloat16)
    a2 = jnp.dot(a1, wg2[...], preferred_element_type=jnp.float32)
    a2 = a2.reshape(Q, K, D) + bg2[...]

    e = jnp.exp(a2 * jnp.float32(1.0 / 16.0))
    o = jnp.sum(e * (vg + pe), axis=1) / jnp.sum(e, axis=1)
    res = jnp.dot(o, w2[...], preferred_element_type=jnp.float32)
    out[...] = res + b2[...] + feat[...]


def _main(g_rows, p_arr, feat_b, Wcat, Wg2, W2, bd1, bd2, bg2, b2):
    grid = (NP // Q,)
    full = lambda q: (0, 0)
    return pl.pallas_call(
        _main_body,
        grid=grid,
        in_specs=[
            pl.BlockSpec((Q * K, HW), lambda q: (q, 0)),
            pl.BlockSpec((Q, 2 * D), lambda q: (q, 0)),
            pl.BlockSpec((Q, D), lambda q: (q, 0)),
            pl.BlockSpec((D, 2 * D), full),
            pl.BlockSpec((D, D), full),
            pl.BlockSpec((D, D), full),
            pl.BlockSpec((1, D), full),
            pl.BlockSpec((1, D), full),
            pl.BlockSpec((1, D), full),
            pl.BlockSpec((1, D), full),
        ],
        out_specs=pl.BlockSpec((Q, D), lambda q: (q, 0)),
        out_shape=jax.ShapeDtypeStruct((N, D), jnp.float32),
    )(g_rows, p_arr, feat_b, Wcat, Wg2, W2,
      bd1.reshape(1, D), bd2.reshape(1, D), bg2.reshape(1, D),
      b2.reshape(1, D))


def kernel(features, pos, pos_center, W1, b1, W2, b2, Wg1, bg1, Wg2, bg2,
           Wd1, bd1, Wd2, bd2, Wbp, bbp, Wq, Wk, Wv):
    pc_pad = jnp.pad(pos_center, [(0, 0), (0, NP - N), (0, 0)])
    pct_pad = jnp.pad(pos_center.transpose(0, 2, 1),
                      [(0, 0), (0, 4), (0, NP - N)])

    Wkg, Wqg, Wcat, Wg2b, cb = _prep(Wk, Wg1, Wq, Wd2, Wg2, bg1, bd2)
    T, P = _pernode(features, pos, W1, b1, Wd1, Wv, Wkg, Wqg, cb)
    wbp2 = Wbp.reshape(2)
    knns, gidxs = [], []
    for b in range(B):
        knn_b, gidx_b = _topk(pc_pad[b], pct_pad[b], wbp2, bbp)
        knns.append(knn_b)
        gidxs.append(gidx_b)

    res = []
    for b in range(B):
        g_b = _sc_gather(T[b], gidxs[b])
        res.append(_main(g_b, P[b], features[b], Wcat, Wg2b, W2,
                         bd1, bd2, bg2, b2))
    return jnp.stack(res, axis=0), jnp.stack(knns, axis=0)
